# Initial kernel scaffold; baseline (speedup 1.0000x reference)
import functools

import jax
import jax.numpy as jnp
from jax import lax
from jax.experimental import pallas as pl
from jax.experimental.pallas import tpu as pltpu
from jax.experimental.pallas import tpu_sc as plsc

N_NODES = 10000
N_EDGES = 320000
NUM_RELS = 5
D = 128

N_TILES = 16
N_SC = 2
B = 128
MAIN_BATCHES = 80
E_PAD = N_SC * N_TILES * MAIN_BATCHES * B
N_PAD_ROWS = 48
DEG_PAD_SLOTS = 48
N_ACC = N_NODES + N_PAD_ROWS
DEG_N = NUM_RELS * N_NODES + DEG_PAD_SLOTS
ROWS_PER_TILE = N_ACC // N_TILES
DEG_PER_TILE = DEG_N // N_TILES
CNT_PER_TILE = E_PAD // N_TILES
CNT_BATCHES = CNT_PER_TILE // B

BLK = 1000


def _transform_body(feat_ref, W0_ref, W1_ref, b_ref, Wrel_ref, T_ref):
    i = pl.program_id(0)
    first_half = i < (5000 // BLK)
    W = jnp.where(first_half, W0_ref[...], W1_ref[...])
    b = jnp.where(first_half, b_ref[0], b_ref[1])
    h = jnp.dot(feat_ref[...], W, preferred_element_type=jnp.float32) + b
    for rr in range(NUM_RELS):
        T_ref[rr] = jnp.dot(h, Wrel_ref[rr], preferred_element_type=jnp.float32)


def _finish_body(p_ref, brel_ref, o_ref):
    o_ref[...] = jnp.maximum(p_ref[0] + p_ref[1] + brel_ref[...], 0.0)


def _sc_agg_body(T_hbm, ru_hbm, rv_hbm, vd_hbm, zrow_hbm, zdeg_hbm, out_hbm,
                 ru_v, rv_v, vd_v, rows_v, deg_v, ones_v, acc_sh, deg_sh, sem):
    c = lax.axis_index("c")
    s = lax.axis_index("s")

    one16 = jnp.ones((16,), jnp.float32)
    for g in range(B // 16):
        ones_v[pl.ds(g * 16, 16)] = one16

    pltpu.sync_copy(zrow_hbm.at[pl.ds(s * ROWS_PER_TILE, ROWS_PER_TILE)],
                    acc_sh.at[pl.ds(s * ROWS_PER_TILE, ROWS_PER_TILE)])
    pltpu.sync_copy(zdeg_hbm.at[pl.ds(s * DEG_PER_TILE, DEG_PER_TILE)],
                    deg_sh.at[pl.ds(s * DEG_PER_TILE, DEG_PER_TILE)])
    plsc.subcore_barrier()

    cnt_base = s * CNT_PER_TILE

    def cnt_body(bi, _):
        base = cnt_base + bi * B
        pltpu.sync_copy(rv_hbm.at[pl.ds(base, B)], rv_v)
        pltpu.sync_copy(ones_v, deg_sh.at[rv_v], add=True)
        return 0

    lax.fori_loop(0, CNT_BATCHES, cnt_body, 0)
    plsc.subcore_barrier()

    w = c * N_TILES + s
    main_base = w * (MAIN_BATCHES * B)

    def main_body(bi, _):
        base = main_base + bi * B
        pltpu.sync_copy(ru_hbm.at[pl.ds(base, B)], ru_v)
        pltpu.sync_copy(rv_hbm.at[pl.ds(base, B)], rv_v)
        pltpu.sync_copy(vd_hbm.at[pl.ds(base, B)], vd_v)
        pltpu.async_copy(T_hbm.at[ru_v], rows_v, sem).wait()
        pltpu.sync_copy(deg_sh.at[rv_v], deg_v)

        def grp(g, _):
            rec = 1.0 / deg_v[pl.ds(g * 16, 16)]
            for l in range(16):
                bc = jnp.take(rec, jnp.full((16,), l, jnp.int32),
                              mode='promise_in_bounds')
                row = g * 16 + l
                for c8 in range(D // 16):
                    sl = pl.ds(c8 * 16, 16)
                    rows_v[row, sl] = rows_v[row, sl] * bc
            return 0

        lax.fori_loop(0, B // 16, grp, 0)
        pltpu.sync_copy(rows_v, acc_sh.at[vd_v], add=True)
        return 0

    lax.fori_loop(0, MAIN_BATCHES, main_body, 0)
    plsc.subcore_barrier()

    pltpu.sync_copy(acc_sh.at[pl.ds(s * ROWS_PER_TILE, ROWS_PER_TILE)],
                    out_hbm.at[c, pl.ds(s * ROWS_PER_TILE, ROWS_PER_TILE)])


_sc_agg = functools.partial(
    pl.kernel,
    out_type=jax.ShapeDtypeStruct((N_SC, N_ACC, D), jnp.float32),
    mesh=plsc.VectorSubcoreMesh(core_axis_name="c", subcore_axis_name="s"),
    scratch_types=[
        pltpu.VMEM((B,), jnp.int32),
        pltpu.VMEM((B,), jnp.int32),
        pltpu.VMEM((B,), jnp.int32),
        pltpu.VMEM((B, D), jnp.float32),
        pltpu.VMEM((B,), jnp.float32),
        pltpu.VMEM((B,), jnp.float32),
        pltpu.VMEM_SHARED((N_ACC, D), jnp.float32),
        pltpu.VMEM_SHARED((DEG_N,), jnp.float32),
        pltpu.SemaphoreType.DMA,
    ],
)(_sc_agg_body)


def kernel(feat0, feat1, edge_index, r, W0, b0, W1, b1, Wrel, brel):
    feat = jnp.concatenate([feat0, feat1], axis=0)
    bstack = jnp.stack([b0, b1])[:, None, :]

    T = pl.pallas_call(
        _transform_body,
        grid=(N_NODES // BLK,),
        in_specs=[
            pl.BlockSpec((BLK, D), lambda i: (i, 0)),
            pl.BlockSpec((D, D), lambda i: (0, 0)),
            pl.BlockSpec((D, D), lambda i: (0, 0)),
            pl.BlockSpec((2, 1, D), lambda i: (0, 0, 0)),
            pl.BlockSpec((NUM_RELS, D, D), lambda i: (0, 0, 0)),
        ],
        out_specs=pl.BlockSpec((NUM_RELS, BLK, D), lambda i: (0, i, 0)),
        out_shape=jax.ShapeDtypeStruct((NUM_RELS, N_NODES, D), jnp.float32),
    )(feat, W0, W1, bstack, Wrel)
    T_flat = T.reshape(NUM_RELS * N_NODES, D)

    u, v = edge_index[0], edge_index[1]
    key_ru = r * N_NODES + u
    key_rv = r * N_NODES + v
    npad = E_PAD - N_EDGES
    pi = jnp.arange(npad, dtype=jnp.int32)
    ru_p = jnp.concatenate([key_ru, pi % 512])
    rv_p = jnp.concatenate([key_rv, NUM_RELS * N_NODES + pi % DEG_PAD_SLOTS])
    vd_p = jnp.concatenate([v, N_NODES + pi % N_PAD_ROWS])

    zrow = jnp.zeros((N_ACC, D), jnp.float32)
    zdeg = jnp.zeros((DEG_N,), jnp.float32)

    partial = _sc_agg(T_flat, ru_p, rv_p, vd_p, zrow, zdeg)

    out = pl.pallas_call(
        _finish_body,
        grid=(N_NODES // BLK,),
        in_specs=[
            pl.BlockSpec((N_SC, BLK, D), lambda i: (0, i, 0)),
            pl.BlockSpec((1, D), lambda i: (0, 0)),
        ],
        out_specs=pl.BlockSpec((BLK, D), lambda i: (i, 0)),
        out_shape=jax.ShapeDtypeStruct((N_NODES, D), jnp.float32),
    )(partial, brel[None, :])
    return out


# speedup vs baseline: 16.4496x; 16.4496x over previous
import functools

import jax
import jax.numpy as jnp
from jax import lax
from jax.experimental import pallas as pl
from jax.experimental.pallas import tpu as pltpu
from jax.experimental.pallas import tpu_sc as plsc

N_NODES = 10000
N_EDGES = 320000
NUM_RELS = 5
D = 128

N_TILES = 16
N_SC = 2
B = 128
MAIN_BATCHES = 80
E_PAD = N_SC * N_TILES * MAIN_BATCHES * B
N_PAD_ROWS = 240
DEG_PAD_SLOTS = 176
N_ACC = N_NODES + N_PAD_ROWS
DEG_N = NUM_RELS * N_NODES + DEG_PAD_SLOTS
ROWS_PER_TILE = N_ACC // N_TILES
DEG_PER_TILE = DEG_N // N_TILES
CNT_PER_TILE = E_PAD // N_TILES
CNT_BATCHES = CNT_PER_TILE // B

BLK = 1000


def _transform_body(feat_ref, W0_ref, W1_ref, b_ref, Wrel_ref, T_ref):
    i = pl.program_id(0)
    first_half = i < (5000 // BLK)
    W = jnp.where(first_half, W0_ref[...], W1_ref[...])
    b = jnp.where(first_half, b_ref[0], b_ref[1])
    h = jnp.dot(feat_ref[...], W, preferred_element_type=jnp.float32) + b
    for rr in range(NUM_RELS):
        T_ref[rr] = jnp.dot(h, Wrel_ref[rr], preferred_element_type=jnp.float32)


def _finish_body(p_ref, brel_ref, o_ref):
    o_ref[...] = jnp.maximum(p_ref[0] + p_ref[1] + brel_ref[...], 0.0)


def _sc_agg_body(T_hbm, ru_hbm, rv_hbm, vd_hbm, out_hbm,
                 ru_v, rv_v, vd_v, rows_v, deg_v, ones_v, zdeg_v,
                 acc_sh, deg_sh, sem):
    c = lax.axis_index("c")
    s = lax.axis_index("s")

    one16 = jnp.ones((16,), jnp.float32)
    zero16 = jnp.zeros((16,), jnp.float32)
    for g in range(B // 16):
        ones_v[pl.ds(g * 16, 16)] = one16

    def zrow_body(i, _):
        for c8 in range(D // 16):
            rows_v[i, pl.ds(c8 * 16, 16)] = zero16
        return 0

    lax.fori_loop(0, B, zrow_body, 0)

    def zdeg_body(i, _):
        zdeg_v[pl.ds(i * 16, 16)] = zero16
        return 0

    lax.fori_loop(0, DEG_PER_TILE // 16, zdeg_body, 0)

    for rep in range(ROWS_PER_TILE // B):
        pltpu.sync_copy(rows_v,
                        acc_sh.at[pl.ds(s * ROWS_PER_TILE + rep * B, B)])
    pltpu.sync_copy(zdeg_v, deg_sh.at[pl.ds(s * DEG_PER_TILE, DEG_PER_TILE)])
    plsc.subcore_barrier()

    cnt_base = s * CNT_PER_TILE

    def cnt_body(bi, _):
        base = cnt_base + bi * B
        pltpu.sync_copy(rv_hbm.at[pl.ds(base, B)], rv_v)
        pltpu.sync_copy(ones_v, deg_sh.at[rv_v], add=True)
        return 0

    lax.fori_loop(0, CNT_BATCHES, cnt_body, 0)
    plsc.subcore_barrier()

    w = c * N_TILES + s
    main_base = w * (MAIN_BATCHES * B)

    def main_body(bi, _):
        base = main_base + bi * B
        pltpu.sync_copy(ru_hbm.at[pl.ds(base, B)], ru_v)
        pltpu.sync_copy(rv_hbm.at[pl.ds(base, B)], rv_v)
        pltpu.sync_copy(vd_hbm.at[pl.ds(base, B)], vd_v)
        pltpu.async_copy(T_hbm.at[ru_v], rows_v, sem).wait()
        pltpu.sync_copy(deg_sh.at[rv_v], deg_v)

        dnums = lax.GatherDimensionNumbers(
            offset_dims=(), collapsed_slice_dims=(0,), start_index_map=(0,))

        def grp(g, _):
            rec = 1.0 / deg_v[pl.ds(g * 16, 16)]
            for l in range(16):
                bc = lax.gather(rec, jnp.full((16, 1), l, jnp.int32),
                                dnums, slice_sizes=(1,),
                                mode=lax.GatherScatterMode.PROMISE_IN_BOUNDS)
                row = g * 16 + l
                for c8 in range(D // 16):
                    sl = pl.ds(c8 * 16, 16)
                    rows_v[row, sl] = rows_v[row, sl] * bc
            return 0

        lax.fori_loop(0, B // 16, grp, 0)
        pltpu.sync_copy(rows_v, acc_sh.at[vd_v], add=True)
        return 0

    lax.fori_loop(0, MAIN_BATCHES, main_body, 0)
    plsc.subcore_barrier()

    pltpu.sync_copy(acc_sh.at[pl.ds(s * ROWS_PER_TILE, ROWS_PER_TILE)],
                    out_hbm.at[c, pl.ds(s * ROWS_PER_TILE, ROWS_PER_TILE)])


_sc_agg = functools.partial(
    pl.kernel,
    out_type=jax.ShapeDtypeStruct((N_SC, N_ACC, D), jnp.float32),
    mesh=plsc.VectorSubcoreMesh(core_axis_name="c", subcore_axis_name="s"),
    scratch_types=[
        pltpu.VMEM((B,), jnp.int32),
        pltpu.VMEM((B,), jnp.int32),
        pltpu.VMEM((B,), jnp.int32),
        pltpu.VMEM((B, D), jnp.float32),
        pltpu.VMEM((B,), jnp.float32),
        pltpu.VMEM((B,), jnp.float32),
        pltpu.VMEM((DEG_PER_TILE,), jnp.float32),
        pltpu.VMEM_SHARED((N_ACC, D), jnp.float32),
        pltpu.VMEM_SHARED((DEG_N,), jnp.float32),
        pltpu.SemaphoreType.DMA,
    ],
)(_sc_agg_body)


def kernel(feat0, feat1, edge_index, r, W0, b0, W1, b1, Wrel, brel):
    feat = jnp.concatenate([feat0, feat1], axis=0)
    bstack = jnp.stack([b0, b1])[:, None, :]

    T = pl.pallas_call(
        _transform_body,
        grid=(N_NODES // BLK,),
        in_specs=[
            pl.BlockSpec((BLK, D), lambda i: (i, 0)),
            pl.BlockSpec((D, D), lambda i: (0, 0)),
            pl.BlockSpec((D, D), lambda i: (0, 0)),
            pl.BlockSpec((2, 1, D), lambda i: (0, 0, 0)),
            pl.BlockSpec((NUM_RELS, D, D), lambda i: (0, 0, 0)),
        ],
        out_specs=pl.BlockSpec((NUM_RELS, BLK, D), lambda i: (0, i, 0)),
        out_shape=jax.ShapeDtypeStruct((NUM_RELS, N_NODES, D), jnp.float32),
    )(feat, W0, W1, bstack, Wrel)
    T_flat = T.reshape(NUM_RELS * N_NODES, D)

    u, v = edge_index[0], edge_index[1]
    key_ru = r * N_NODES + u
    key_rv = r * N_NODES + v
    npad = E_PAD - N_EDGES
    pi = jnp.arange(npad, dtype=jnp.int32)
    ru_p = jnp.concatenate([key_ru, pi % 512])
    rv_p = jnp.concatenate([key_rv, NUM_RELS * N_NODES + pi % DEG_PAD_SLOTS])
    vd_p = jnp.concatenate([v, N_NODES + pi % N_PAD_ROWS])

    partial = _sc_agg(T_flat, ru_p, rv_p, vd_p)

    out = pl.pallas_call(
        _finish_body,
        grid=(N_NODES // BLK,),
        in_specs=[
            pl.BlockSpec((N_SC, BLK, D), lambda i: (0, i, 0)),
            pl.BlockSpec((1, D), lambda i: (0, 0)),
        ],
        out_specs=pl.BlockSpec((BLK, D), lambda i: (i, 0)),
        out_shape=jax.ShapeDtypeStruct((N_NODES, D), jnp.float32),
    )(partial, brel[None, :])
    return out


# speedup vs baseline: 29.0567x; 1.7664x over previous
import functools

import jax
import jax.numpy as jnp
from jax import lax
from jax.experimental import pallas as pl
from jax.experimental.pallas import tpu as pltpu
from jax.experimental.pallas import tpu_sc as plsc

N_NODES = 10000
N_EDGES = 320000
NUM_RELS = 5
D = 128

N_TILES = 16
N_SC = 2
B = 128
MAIN_BATCHES = 80
E_PAD = N_SC * N_TILES * MAIN_BATCHES * B
NB_TOTAL = E_PAD // B
N_PAD_ROWS = 240
DEG_PAD_SLOTS = 176
N_ACC = N_NODES + N_PAD_ROWS
DEG_N = NUM_RELS * N_NODES + DEG_PAD_SLOTS
ROWS_PER_TILE = N_ACC // N_TILES
DEG_PER_TILE = DEG_N // N_TILES
CNT_BATCHES = NB_TOTAL // N_TILES

BLK = 1000


def _transform_body(feat_ref, W0_ref, W1_ref, b_ref, Wrel_ref, T_ref):
    i = pl.program_id(0)
    first_half = i < (5000 // BLK)
    W = jnp.where(first_half, W0_ref[...], W1_ref[...])
    b = jnp.where(first_half, b_ref[0], b_ref[1])
    h = jnp.dot(feat_ref[...], W, preferred_element_type=jnp.float32) + b
    for rr in range(NUM_RELS):
        T_ref[rr] = jnp.dot(h, Wrel_ref[rr], preferred_element_type=jnp.float32)


def _finish_body(p_ref, brel_ref, o_ref):
    o_ref[...] = jnp.maximum(p_ref[0] + p_ref[1] + brel_ref[...], 0.0)


def _sc_agg_body(T_hbm, pk_hbm, out_hbm,
                 pk_a, pk_b, rows_a, rows_b, deg_a, deg_b, ones_v, zdeg_v,
                 acc_sh, deg_sh,
                 spk_a, spk_b, srow_a, srow_b, sdeg_a, sdeg_b, ssc_a, ssc_b):
    c = lax.axis_index("c")
    s = lax.axis_index("s")

    pk = (pk_a, pk_b)
    rows = (rows_a, rows_b)
    deg = (deg_a, deg_b)
    spk = (spk_a, spk_b)
    srow = (srow_a, srow_b)
    sdeg = (sdeg_a, sdeg_b)
    ssc = (ssc_a, ssc_b)

    one16 = jnp.ones((16,), jnp.float32)
    zero16 = jnp.zeros((16,), jnp.float32)
    for g in range(B // 16):
        ones_v[pl.ds(g * 16, 16)] = one16

    def zrow_body(i, _):
        for c8 in range(D // 16):
            rows_a[i, pl.ds(c8 * 16, 16)] = zero16
        return 0

    lax.fori_loop(0, B, zrow_body, 0)

    def zdeg_body(i, _):
        zdeg_v[pl.ds(i * 16, 16)] = zero16
        return 0

    lax.fori_loop(0, DEG_PER_TILE // 16, zdeg_body, 0)

    for rep in range(ROWS_PER_TILE // B):
        pltpu.sync_copy(rows_a,
                        acc_sh.at[pl.ds(s * ROWS_PER_TILE + rep * B, B)])
    pltpu.sync_copy(zdeg_v, deg_sh.at[pl.ds(s * DEG_PER_TILE, DEG_PER_TILE)])
    plsc.subcore_barrier()

    def cnt_start_pack(k, st):
        pltpu.async_copy(pk_hbm.at[s * CNT_BATCHES + k], pk[st], spk[st])

    def cnt_wait_pack(k, st):
        pltpu.make_async_copy(pk_hbm.at[s * CNT_BATCHES + k], pk[st],
                              spk[st]).wait()

    def cnt_half(k, st):
        ot = 1 - st

        @pl.when(k + 1 < CNT_BATCHES)
        def _():
            cnt_start_pack(k + 1, ot)

        cnt_wait_pack(k, st)
        pltpu.sync_copy(ones_v, deg_sh.at[pk[st].at[1]], add=True)

    cnt_start_pack(0, 0)

    def cnt_body(j, _):
        cnt_half(2 * j, 0)
        cnt_half(2 * j + 1, 1)
        return 0

    lax.fori_loop(0, CNT_BATCHES // 2, cnt_body, 0)
    plsc.subcore_barrier()

    w = c * N_TILES + s

    dnums = lax.GatherDimensionNumbers(
        offset_dims=(), collapsed_slice_dims=(0,), start_index_map=(0,))

    def m_start_pack(i, st):
        pltpu.async_copy(pk_hbm.at[w * MAIN_BATCHES + i], pk[st], spk[st])

    def m_wait_pack(i, st):
        pltpu.make_async_copy(pk_hbm.at[w * MAIN_BATCHES + i], pk[st],
                              spk[st]).wait()

    def m_start_gathers(st):
        pltpu.async_copy(T_hbm.at[pk[st].at[0]], rows[st], srow[st])
        pltpu.async_copy(deg_sh.at[pk[st].at[1]], deg[st], sdeg[st])

    def m_wait_gathers(st):
        pltpu.make_async_copy(T_hbm.at[pk[st].at[0]], rows[st],
                              srow[st]).wait()
        pltpu.make_async_copy(deg_sh.at[pk[st].at[1]], deg[st],
                              sdeg[st]).wait()

    def m_scale(st):
        rows_st, deg_st = rows[st], deg[st]

        def grp(g, _):
            rec = 1.0 / deg_st[pl.ds(g * 16, 16)]
            for l in range(16):
                bc = lax.gather(rec, jnp.full((16, 1), l, jnp.int32),
                                dnums, slice_sizes=(1,),
                                mode=lax.GatherScatterMode.PROMISE_IN_BOUNDS)
                row = g * 16 + l
                for c8 in range(D // 16):
                    sl = pl.ds(c8 * 16, 16)
                    rows_st[row, sl] = rows_st[row, sl] * bc
            return 0

        lax.fori_loop(0, B // 16, grp, 0)

    def m_half(i, st):
        ot = 1 - st

        @pl.when(i + 1 < MAIN_BATCHES)
        def _():
            m_start_pack(i + 1, ot)

        m_wait_gathers(st)

        @pl.when(i + 1 < MAIN_BATCHES)
        def _():
            m_wait_pack(i + 1, ot)
            m_start_gathers(ot)

        m_scale(st)
        pltpu.sync_copy(rows[st], acc_sh.at[pk[st].at[2]], add=True)

    m_start_pack(0, 0)
    m_wait_pack(0, 0)
    m_start_gathers(0)

    def m_body(j, _):
        m_half(2 * j, 0)
        m_half(2 * j + 1, 1)
        return 0

    lax.fori_loop(0, MAIN_BATCHES // 2, m_body, 0)
    plsc.subcore_barrier()

    pltpu.sync_copy(acc_sh.at[pl.ds(s * ROWS_PER_TILE, ROWS_PER_TILE)],
                    out_hbm.at[c, pl.ds(s * ROWS_PER_TILE, ROWS_PER_TILE)])


_sc_agg = functools.partial(
    pl.kernel,
    out_type=jax.ShapeDtypeStruct((N_SC, N_ACC, D), jnp.float32),
    mesh=plsc.VectorSubcoreMesh(core_axis_name="c", subcore_axis_name="s"),
    scratch_types=[
        pltpu.VMEM((3, B), jnp.int32),
        pltpu.VMEM((3, B), jnp.int32),
        pltpu.VMEM((B, D), jnp.float32),
        pltpu.VMEM((B, D), jnp.float32),
        pltpu.VMEM((B,), jnp.float32),
        pltpu.VMEM((B,), jnp.float32),
        pltpu.VMEM((B,), jnp.float32),
        pltpu.VMEM((DEG_PER_TILE,), jnp.float32),
        pltpu.VMEM_SHARED((N_ACC, D), jnp.float32),
        pltpu.VMEM_SHARED((DEG_N,), jnp.float32),
        pltpu.SemaphoreType.DMA,
        pltpu.SemaphoreType.DMA,
        pltpu.SemaphoreType.DMA,
        pltpu.SemaphoreType.DMA,
        pltpu.SemaphoreType.DMA,
        pltpu.SemaphoreType.DMA,
        pltpu.SemaphoreType.DMA,
        pltpu.SemaphoreType.DMA,
    ],
)(_sc_agg_body)


def kernel(feat0, feat1, edge_index, r, W0, b0, W1, b1, Wrel, brel):
    feat = jnp.concatenate([feat0, feat1], axis=0)
    bstack = jnp.stack([b0, b1])[:, None, :]

    T = pl.pallas_call(
        _transform_body,
        grid=(N_NODES // BLK,),
        in_specs=[
            pl.BlockSpec((BLK, D), lambda i: (i, 0)),
            pl.BlockSpec((D, D), lambda i: (0, 0)),
            pl.BlockSpec((D, D), lambda i: (0, 0)),
            pl.BlockSpec((2, 1, D), lambda i: (0, 0, 0)),
            pl.BlockSpec((NUM_RELS, D, D), lambda i: (0, 0, 0)),
        ],
        out_specs=pl.BlockSpec((NUM_RELS, BLK, D), lambda i: (0, i, 0)),
        out_shape=jax.ShapeDtypeStruct((NUM_RELS, N_NODES, D), jnp.float32),
    )(feat, W0, W1, bstack, Wrel)
    T_flat = T.reshape(NUM_RELS * N_NODES, D)

    u, v = edge_index[0], edge_index[1]
    key_ru = r * N_NODES + u
    key_rv = r * N_NODES + v
    npad = E_PAD - N_EDGES
    pi = jnp.arange(npad, dtype=jnp.int32)
    ru_p = jnp.concatenate([key_ru, pi % 512])
    rv_p = jnp.concatenate([key_rv, NUM_RELS * N_NODES + pi % DEG_PAD_SLOTS])
    vd_p = jnp.concatenate([v, N_NODES + pi % N_PAD_ROWS])
    pack = jnp.stack([ru_p.reshape(NB_TOTAL, B), rv_p.reshape(NB_TOTAL, B),
                      vd_p.reshape(NB_TOTAL, B)], axis=1)

    partial = _sc_agg(T_flat, pack)

    out = pl.pallas_call(
        _finish_body,
        grid=(N_NODES // BLK,),
        in_specs=[
            pl.BlockSpec((N_SC, BLK, D), lambda i: (0, i, 0)),
            pl.BlockSpec((1, D), lambda i: (0, 0)),
        ],
        out_specs=pl.BlockSpec((BLK, D), lambda i: (i, 0)),
        out_shape=jax.ShapeDtypeStruct((N_NODES, D), jnp.float32),
    )(partial, brel[None, :])
    return out


# speedup vs baseline: 34.2081x; 1.1773x over previous
import functools

import jax
import jax.numpy as jnp
from jax import lax
from jax.experimental import pallas as pl
from jax.experimental.pallas import tpu as pltpu
from jax.experimental.pallas import tpu_sc as plsc

N_NODES = 10000
N_EDGES = 320000
NUM_RELS = 5
D = 128

N_TILES = 16
N_SC = 2
B = 128
MAIN_BATCHES = 80
E_PAD = N_SC * N_TILES * MAIN_BATCHES * B
NB_TOTAL = E_PAD // B
N_PAD_ROWS = 240
DEG_PAD_SLOTS = 176
N_ACC = N_NODES + N_PAD_ROWS
DEG_N = NUM_RELS * N_NODES + DEG_PAD_SLOTS
ROWS_PER_TILE = N_ACC // N_TILES
DEG_PER_TILE = DEG_N // N_TILES

BLK = 1000


def _transform_body(feat_ref, W0_ref, W1_ref, b_ref, Wrel_ref, T_ref):
    i = pl.program_id(0)
    first_half = i < (5000 // BLK)
    W = jnp.where(first_half, W0_ref[...], W1_ref[...])
    b = jnp.where(first_half, b_ref[0], b_ref[1])
    h = jnp.dot(feat_ref[...], W, preferred_element_type=jnp.float32) + b
    for rr in range(NUM_RELS):
        T_ref[rr] = jnp.dot(h, Wrel_ref[rr], preferred_element_type=jnp.float32)


def _finish_body(p_ref, brel_ref, o_ref):
    o_ref[...] = jnp.maximum(p_ref[0] + p_ref[1] + brel_ref[...], 0.0)


def _recip_body(p_ref, o_ref):
    d = p_ref[0] + p_ref[1]
    o_ref[...] = jnp.where(d > 0, 1.0 / jnp.maximum(d, 1.0), 0.0)


def _sc_count_body(pk_hbm, degp_hbm,
                   pk_a, pk_b, ones_v, zdeg_v, deg_sh, spk_a, spk_b):
    c = lax.axis_index("c")
    s = lax.axis_index("s")
    w = c * N_TILES + s

    pk = (pk_a, pk_b)
    spk = (spk_a, spk_b)

    one16 = jnp.ones((16,), jnp.float32)
    zero16 = jnp.zeros((16,), jnp.float32)
    for g in range(B // 16):
        ones_v[pl.ds(g * 16, 16)] = one16

    def zdeg_body(i, _):
        zdeg_v[pl.ds(i * 16, 16)] = zero16
        return 0

    lax.fori_loop(0, DEG_PER_TILE // 16, zdeg_body, 0)
    pltpu.sync_copy(zdeg_v, deg_sh.at[pl.ds(s * DEG_PER_TILE, DEG_PER_TILE)])
    plsc.subcore_barrier()

    def cnt_start_pack(k, st):
        pltpu.async_copy(pk_hbm.at[w * MAIN_BATCHES + k], pk[st], spk[st])

    def cnt_wait_pack(k, st):
        pltpu.make_async_copy(pk_hbm.at[w * MAIN_BATCHES + k], pk[st],
                              spk[st]).wait()

    def cnt_half(k, st):
        ot = 1 - st

        @pl.when(k + 1 < MAIN_BATCHES)
        def _():
            cnt_start_pack(k + 1, ot)

        cnt_wait_pack(k, st)
        pltpu.sync_copy(ones_v, deg_sh.at[pk[st].at[1]], add=True)

    cnt_start_pack(0, 0)

    def cnt_body(j, _):
        cnt_half(2 * j, 0)
        cnt_half(2 * j + 1, 1)
        return 0

    lax.fori_loop(0, MAIN_BATCHES // 2, cnt_body, 0)
    plsc.subcore_barrier()

    pltpu.sync_copy(deg_sh.at[pl.ds(s * DEG_PER_TILE, DEG_PER_TILE)], zdeg_v)
    pltpu.sync_copy(zdeg_v,
                    degp_hbm.at[pl.ds(c * DEG_N + s * DEG_PER_TILE,
                                      DEG_PER_TILE)])


_sc_count = functools.partial(
    pl.kernel,
    out_type=jax.ShapeDtypeStruct((N_SC * DEG_N,), jnp.float32),
    mesh=plsc.VectorSubcoreMesh(core_axis_name="c", subcore_axis_name="s"),
    scratch_types=[
        pltpu.VMEM((3, B), jnp.int32),
        pltpu.VMEM((3, B), jnp.int32),
        pltpu.VMEM((B,), jnp.float32),
        pltpu.VMEM((DEG_PER_TILE,), jnp.float32),
        pltpu.VMEM_SHARED((DEG_N,), jnp.float32),
        pltpu.SemaphoreType.DMA,
        pltpu.SemaphoreType.DMA,
    ],
)(_sc_count_body)


def _sc_agg_body(T_hbm, pk_hbm, rec_hbm, out_hbm,
                 pk_a, pk_b, rows_a, rows_b, rec_a, rec_b, acc_sh,
                 spk_a, spk_b, srow_a, srow_b, sdeg_a, sdeg_b):
    c = lax.axis_index("c")
    s = lax.axis_index("s")

    pk = (pk_a, pk_b)
    rows = (rows_a, rows_b)
    rec = (rec_a, rec_b)
    spk = (spk_a, spk_b)
    srow = (srow_a, srow_b)
    sdeg = (sdeg_a, sdeg_b)

    zero16 = jnp.zeros((16,), jnp.float32)

    def zrow_body(i, _):
        for c8 in range(D // 16):
            rows_a[i, pl.ds(c8 * 16, 16)] = zero16
        return 0

    lax.fori_loop(0, B, zrow_body, 0)

    for rep in range(ROWS_PER_TILE // B):
        pltpu.sync_copy(rows_a,
                        acc_sh.at[pl.ds(s * ROWS_PER_TILE + rep * B, B)])
    plsc.subcore_barrier()

    w = c * N_TILES + s

    dnums = lax.GatherDimensionNumbers(
        offset_dims=(), collapsed_slice_dims=(0,), start_index_map=(0,))

    def m_start_pack(i, st):
        pltpu.async_copy(pk_hbm.at[w * MAIN_BATCHES + i], pk[st], spk[st])

    def m_wait_pack(i, st):
        pltpu.make_async_copy(pk_hbm.at[w * MAIN_BATCHES + i], pk[st],
                              spk[st]).wait()

    def m_start_gathers(st):
        pltpu.async_copy(T_hbm.at[pk[st].at[0]], rows[st], srow[st])
        pltpu.async_copy(rec_hbm.at[pk[st].at[1]], rec[st], sdeg[st])

    def m_wait_gathers(st):
        pltpu.make_async_copy(T_hbm.at[pk[st].at[0]], rows[st],
                              srow[st]).wait()
        pltpu.make_async_copy(rec_hbm.at[pk[st].at[1]], rec[st],
                              sdeg[st]).wait()

    def m_scale(st):
        rows_st, rec_st = rows[st], rec[st]

        def grp(g, _):
            r16 = rec_st[pl.ds(g * 16, 16)]
            for l in range(16):
                bc = lax.gather(r16, jnp.full((16, 1), l, jnp.int32),
                                dnums, slice_sizes=(1,),
                                mode=lax.GatherScatterMode.PROMISE_IN_BOUNDS)
                row = g * 16 + l
                for c8 in range(D // 16):
                    sl = pl.ds(c8 * 16, 16)
                    rows_st[row, sl] = rows_st[row, sl] * bc
            return 0

        lax.fori_loop(0, B // 16, grp, 0)

    def m_half(i, st):
        ot = 1 - st

        @pl.when(i + 1 < MAIN_BATCHES)
        def _():
            m_start_pack(i + 1, ot)

        m_wait_gathers(st)

        @pl.when(i + 1 < MAIN_BATCHES)
        def _():
            m_wait_pack(i + 1, ot)
            m_start_gathers(ot)

        m_scale(st)
        pltpu.sync_copy(rows[st], acc_sh.at[pk[st].at[2]], add=True)

    m_start_pack(0, 0)
    m_wait_pack(0, 0)
    m_start_gathers(0)

    def m_body(j, _):
        m_half(2 * j, 0)
        m_half(2 * j + 1, 1)
        return 0

    lax.fori_loop(0, MAIN_BATCHES // 2, m_body, 0)
    plsc.subcore_barrier()

    pltpu.sync_copy(acc_sh.at[pl.ds(s * ROWS_PER_TILE, ROWS_PER_TILE)],
                    out_hbm.at[c, pl.ds(s * ROWS_PER_TILE, ROWS_PER_TILE)])


_sc_agg = functools.partial(
    pl.kernel,
    out_type=jax.ShapeDtypeStruct((N_SC, N_ACC, D), jnp.float32),
    mesh=plsc.VectorSubcoreMesh(core_axis_name="c", subcore_axis_name="s"),
    scratch_types=[
        pltpu.VMEM((3, B), jnp.int32),
        pltpu.VMEM((3, B), jnp.int32),
        pltpu.VMEM((B, D), jnp.float32),
        pltpu.VMEM((B, D), jnp.float32),
        pltpu.VMEM((B,), jnp.float32),
        pltpu.VMEM((B,), jnp.float32),
        pltpu.VMEM_SHARED((N_ACC, D), jnp.float32),
        pltpu.SemaphoreType.DMA,
        pltpu.SemaphoreType.DMA,
        pltpu.SemaphoreType.DMA,
        pltpu.SemaphoreType.DMA,
        pltpu.SemaphoreType.DMA,
        pltpu.SemaphoreType.DMA,
    ],
)(_sc_agg_body)


def kernel(feat0, feat1, edge_index, r, W0, b0, W1, b1, Wrel, brel):
    feat = jnp.concatenate([feat0, feat1], axis=0)
    bstack = jnp.stack([b0, b1])[:, None, :]

    T = pl.pallas_call(
        _transform_body,
        grid=(N_NODES // BLK,),
        in_specs=[
            pl.BlockSpec((BLK, D), lambda i: (i, 0)),
            pl.BlockSpec((D, D), lambda i: (0, 0)),
            pl.BlockSpec((D, D), lambda i: (0, 0)),
            pl.BlockSpec((2, 1, D), lambda i: (0, 0, 0)),
            pl.BlockSpec((NUM_RELS, D, D), lambda i: (0, 0, 0)),
        ],
        out_specs=pl.BlockSpec((NUM_RELS, BLK, D), lambda i: (0, i, 0)),
        out_shape=jax.ShapeDtypeStruct((NUM_RELS, N_NODES, D), jnp.float32),
    )(feat, W0, W1, bstack, Wrel)
    T_flat = T.reshape(NUM_RELS * N_NODES, D)

    u, v = edge_index[0], edge_index[1]
    key_ru = r * N_NODES + u
    key_rv = r * N_NODES + v
    npad = E_PAD - N_EDGES
    pi = jnp.arange(npad, dtype=jnp.int32)
    ru_p = jnp.concatenate([key_ru, pi % 512])
    rv_p = jnp.concatenate([key_rv, NUM_RELS * N_NODES + pi % DEG_PAD_SLOTS])
    vd_p = jnp.concatenate([v, N_NODES + pi % N_PAD_ROWS])
    pack = jnp.stack([ru_p.reshape(NB_TOTAL, B), rv_p.reshape(NB_TOTAL, B),
                      vd_p.reshape(NB_TOTAL, B)], axis=1)

    degp = _sc_count(pack)
    rec_table = pl.pallas_call(
        _recip_body,
        grid=(1,),
        in_specs=[pl.BlockSpec((N_SC, DEG_N // D, D), lambda i: (0, 0, 0))],
        out_specs=pl.BlockSpec((DEG_N // D, D), lambda i: (0, 0)),
        out_shape=jax.ShapeDtypeStruct((DEG_N // D, D), jnp.float32),
    )(degp.reshape(N_SC, DEG_N // D, D)).reshape(DEG_N)

    partial = _sc_agg(T_flat, pack, rec_table)

    out = pl.pallas_call(
        _finish_body,
        grid=(N_NODES // BLK,),
        in_specs=[
            pl.BlockSpec((N_SC, BLK, D), lambda i: (0, i, 0)),
            pl.BlockSpec((1, D), lambda i: (0, 0)),
        ],
        out_specs=pl.BlockSpec((BLK, D), lambda i: (i, 0)),
        out_shape=jax.ShapeDtypeStruct((N_NODES, D), jnp.float32),
    )(partial, brel[None, :])
    return out
